# baseline (speedup 1.0000x reference)
import jax
import jax.numpy as jnp
from jax import lax
from jax.experimental import pallas as pl
from jax.experimental.pallas import tpu as pltpu
from jax.experimental.pallas import tpu_sc as plsc

NUM_CORES = 2
NUM_SUBCORES = 16
LANES = 16
NW = NUM_CORES * NUM_SUBCORES

NROWS = 1000000
EMBED_DIM = 64
PAIR_DIM = 2 * EMBED_DIM
BATCH = 16384
B_PER_W = BATCH // NW
ICHUNK = 128
N_ICHUNKS = B_PER_W // ICHUNK
CCHUNK = LANES
N_CCHUNKS = B_PER_W // CCHUNK

TCOLS = 16384
HALF = TCOLS // 2
QUART = TCOLS // 4
BLK_SHIFT = 14
HALF_SHIFT = 13
QUART_SHIFT = 12
TGRID = -(-NROWS // TCOLS)
G_ROWS = TGRID * QUART


def _tc_repack(tt_ref, out_ref):
    x = tt_ref[...]
    u = lax.bitcast_convert_type(x, jnp.int32)
    ur = u + jnp.int32(0x8000)
    lo = (lax.slice(ur, (0, 0), (32, TCOLS)) >> 16) & jnp.int32(0xFFFF)
    hi = lax.slice(ur, (32, 0), (64, TCOLS)) & jnp.int32(-65536)
    w = lo | hi
    w4 = jnp.concatenate(
        [lax.slice(w, (0, q * QUART), (32, (q + 1) * QUART))
         for q in range(4)], axis=0)
    out_ref[...] = lax.transpose(w4, (1, 0))


def _repack_table(tablet):
    return pl.pallas_call(
        _tc_repack,
        grid=(TGRID,),
        in_specs=[pl.BlockSpec((EMBED_DIM, TCOLS), lambda i: (0, i))],
        out_specs=pl.BlockSpec((QUART, PAIR_DIM), lambda i: (i, 0)),
        out_shape=jax.ShapeDtypeStruct((G_ROWS, PAIR_DIM), jnp.int32),
        compiler_params=pltpu.CompilerParams(
            dimension_semantics=("arbitrary",)),
    )(tablet)


def _rsqrt(x):
    i = plsc.bitcast(x, jnp.int32)
    y = plsc.bitcast(jnp.int32(0x5F3759DF) - (i >> 1), jnp.float32)
    for _ in range(3):
        y = y * (jnp.float32(1.5) - jnp.float32(0.5) * x * y * y)
    return y


def _gather_descs(table_hbm, hi_v, rows_v, gsem):
    for k in range(N_ICHUNKS):
        yield pltpu.make_async_copy(
            table_hbm.at[hi_v.at[k]],
            rows_v.at[pl.ds(k * ICHUNK, ICHUNK)], gsem)


def _sc_body(table_hbm, nodes_hbm, out_hbm, idx_v, hi_v, rows_v, t_v, gsem):
    wid = lax.axis_index("s") * NUM_CORES + lax.axis_index("c")
    base = wid * B_PER_W

    pltpu.sync_copy(nodes_hbm.at[wid], idx_v)
    for k in range(N_ICHUNKS):
        for j in range(ICHUNK // LANES):
            sl = pl.ds(j * LANES, LANES)
            iv = idx_v[k, sl]
            hi_v[k, sl] = ((iv >> BLK_SHIFT) << QUART_SHIFT) + \
                (iv & (QUART - 1))

    for c in _gather_descs(table_hbm, hi_v, rows_v, gsem):
        c.start()

    lane = lax.broadcasted_iota(jnp.int32, (LANES,), 0)
    himask = jnp.int32(-65536)

    def chunk_body(c):
        row = c * CCHUNK + lane
        iv = plsc.load_gather(idx_v, [row >> 7, row & (ICHUNK - 1)])
        wb = ((iv >> QUART_SHIFT) & 3) * 32
        acc = jnp.zeros((LANES,), jnp.float32)
        for dp in range(EMBED_DIM // 2):
            w = plsc.load_gather(rows_v, [row, wb + dp])
            ve = plsc.bitcast(w << 16, jnp.float32)
            vo = plsc.bitcast(w & himask, jnp.float32)
            acc = acc + ve * ve + vo * vo
        r = _rsqrt(acc)
        for dp in range(EMBED_DIM // 2):
            w = plsc.load_gather(rows_v, [row, wb + dp])
            ve = plsc.bitcast(w << 16, jnp.float32)
            vo = plsc.bitcast(w & himask, jnp.float32)
            t_v[dp, pl.ds(c * CCHUNK, CCHUNK)] = ve * r
            t_v[dp + 32, pl.ds(c * CCHUNK, CCHUNK)] = vo * r

    per_k = N_CCHUNKS // N_ICHUNKS
    for k, c in enumerate(_gather_descs(table_hbm, hi_v, rows_v, gsem)):
        c.wait()
        pl.loop(k * per_k, (k + 1) * per_k)(chunk_body)

    pltpu.sync_copy(t_v, out_hbm.at[:, pl.ds(base, B_PER_W)])


@jax.jit
def _encode(nodes, table):
    nodes_r = nodes.astype(jnp.int32).reshape(NW, N_ICHUNKS, ICHUNK)
    table_p = _repack_table(table.T)
    mesh = plsc.VectorSubcoreMesh(core_axis_name="c", subcore_axis_name="s")
    return pl.kernel(
        _sc_body,
        out_type=jax.ShapeDtypeStruct((EMBED_DIM, BATCH), jnp.float32),
        mesh=mesh,
        compiler_params=pltpu.CompilerParams(needs_layout_passes=False),
        scratch_types=[
            pltpu.VMEM((N_ICHUNKS, ICHUNK), jnp.int32),
            pltpu.VMEM((N_ICHUNKS, ICHUNK), jnp.int32),
            pltpu.VMEM((B_PER_W, PAIR_DIM), jnp.int32),
            pltpu.VMEM((EMBED_DIM, B_PER_W), jnp.float32),
            pltpu.SemaphoreType.DMA,
        ],
    )(table_p, nodes_r)


def kernel(nodes, table):
    return _encode(nodes, table)


# speedup vs baseline: 1.0333x; 1.0333x over previous
import jax
import jax.numpy as jnp
from jax import lax
from jax.experimental import pallas as pl
from jax.experimental.pallas import tpu as pltpu
from jax.experimental.pallas import tpu_sc as plsc

NUM_CORES = 2
NUM_SUBCORES = 16
LANES = 16
NW = NUM_CORES * NUM_SUBCORES

NROWS = 1000000
EMBED_DIM = 64
PAIR_DIM = 2 * EMBED_DIM
BATCH = 16384
B_PER_W = BATCH // NW
ICHUNK = 128
N_ICHUNKS = B_PER_W // ICHUNK
CCHUNK = LANES
N_CCHUNKS = B_PER_W // CCHUNK

TCOLS = 16384
HALF = TCOLS // 2
QUART = TCOLS // 4
BLK_SHIFT = 14
HALF_SHIFT = 13
QUART_SHIFT = 12
TGRID = -(-NROWS // TCOLS)
G_ROWS = TGRID * QUART


def _tc_repack(tt_ref, out_ref):
    x = tt_ref[...]
    u = lax.bitcast_convert_type(x, jnp.int32)
    ur = u + jnp.int32(0x8000)
    lo = (lax.slice(ur, (0, 0), (32, TCOLS)) >> 16) & jnp.int32(0xFFFF)
    hi = lax.slice(ur, (32, 0), (64, TCOLS)) & jnp.int32(-65536)
    w = lo | hi
    w4 = jnp.concatenate(
        [lax.slice(w, (0, q * QUART), (32, (q + 1) * QUART))
         for q in range(4)], axis=0)
    out_ref[...] = lax.transpose(w4, (1, 0))


def _repack_table(tablet):
    return pl.pallas_call(
        _tc_repack,
        grid=(TGRID,),
        in_specs=[pl.BlockSpec((EMBED_DIM, TCOLS), lambda i: (0, i))],
        out_specs=pl.BlockSpec((QUART, PAIR_DIM), lambda i: (i, 0)),
        out_shape=jax.ShapeDtypeStruct((G_ROWS, PAIR_DIM), jnp.int32),
        compiler_params=pltpu.CompilerParams(
            dimension_semantics=("arbitrary",)),
    )(tablet)


def _rsqrt(x):
    i = plsc.bitcast(x, jnp.int32)
    y = plsc.bitcast(jnp.int32(0x5F3759DF) - (i >> 1), jnp.float32)
    for _ in range(3):
        y = y * (jnp.float32(1.5) - jnp.float32(0.5) * x * y * y)
    return y


def _gather_descs(table_hbm, hi_v, rows_v, gsem):
    for k in range(N_ICHUNKS):
        yield pltpu.make_async_copy(
            table_hbm.at[hi_v.at[k]],
            rows_v.at[pl.ds(k * ICHUNK, ICHUNK)], gsem)


def _sc_body(table_hbm, nodes_hbm, out_hbm, idx_v, hi_v, rows_v, t_v, gsem):
    wid = lax.axis_index("s") * NUM_CORES + lax.axis_index("c")
    base = wid * B_PER_W

    pltpu.sync_copy(nodes_hbm.at[wid], idx_v)
    for k in range(N_ICHUNKS):
        for j in range(ICHUNK // LANES):
            sl = pl.ds(j * LANES, LANES)
            iv = idx_v[k, sl]
            hi_v[k, sl] = ((iv >> BLK_SHIFT) << QUART_SHIFT) + \
                (iv & (QUART - 1))

    for c in _gather_descs(table_hbm, hi_v, rows_v, gsem):
        c.start()
    for c in _gather_descs(table_hbm, hi_v, rows_v, gsem):
        c.wait()

    lane = lax.broadcasted_iota(jnp.int32, (LANES,), 0)
    himask = jnp.int32(-65536)

    def chunk_body(c):
        row = c * CCHUNK + lane
        iv = plsc.load_gather(idx_v, [row >> 7, row & (ICHUNK - 1)])
        wb = ((iv >> QUART_SHIFT) & 3) * 32
        acc = jnp.zeros((LANES,), jnp.float32)
        for dp in range(EMBED_DIM // 2):
            w = plsc.load_gather(rows_v, [row, wb + dp])
            ve = plsc.bitcast(w << 16, jnp.float32)
            vo = plsc.bitcast(w & himask, jnp.float32)
            acc = acc + ve * ve + vo * vo
            t_v[dp, pl.ds(c * CCHUNK, CCHUNK)] = ve
            t_v[dp + 32, pl.ds(c * CCHUNK, CCHUNK)] = vo
        r = _rsqrt(acc)
        for d in range(EMBED_DIM):
            sl = pl.ds(c * CCHUNK, CCHUNK)
            t_v[d, sl] = t_v[d, sl] * r

    pl.loop(0, N_CCHUNKS)(chunk_body)

    pltpu.sync_copy(t_v, out_hbm.at[:, pl.ds(base, B_PER_W)])


@jax.jit
def _encode(nodes, table):
    nodes_r = nodes.astype(jnp.int32).reshape(NW, N_ICHUNKS, ICHUNK)
    table_p = _repack_table(table.T)
    mesh = plsc.VectorSubcoreMesh(core_axis_name="c", subcore_axis_name="s")
    return pl.kernel(
        _sc_body,
        out_type=jax.ShapeDtypeStruct((EMBED_DIM, BATCH), jnp.float32),
        mesh=mesh,
        compiler_params=pltpu.CompilerParams(needs_layout_passes=False),
        scratch_types=[
            pltpu.VMEM((N_ICHUNKS, ICHUNK), jnp.int32),
            pltpu.VMEM((N_ICHUNKS, ICHUNK), jnp.int32),
            pltpu.VMEM((B_PER_W, PAIR_DIM), jnp.int32),
            pltpu.VMEM((EMBED_DIM, B_PER_W), jnp.float32),
            pltpu.SemaphoreType.DMA,
        ],
    )(table_p, nodes_r)


def kernel(nodes, table):
    return _encode(nodes, table)


# speedup vs baseline: 1.0334x; 1.0001x over previous
import jax
import jax.numpy as jnp
from jax import lax
from jax.experimental import pallas as pl
from jax.experimental.pallas import tpu as pltpu
from jax.experimental.pallas import tpu_sc as plsc

NUM_CORES = 2
NUM_SUBCORES = 16
LANES = 16
NW = NUM_CORES * NUM_SUBCORES

NROWS = 1000000
EMBED_DIM = 64
PAIR_DIM = 2 * EMBED_DIM
BATCH = 16384
B_PER_W = BATCH // NW
ICHUNK = 128
N_ICHUNKS = B_PER_W // ICHUNK
CCHUNK = LANES
N_CCHUNKS = B_PER_W // CCHUNK

TCOLS = 16384
QUART = TCOLS // 4
BLK_SHIFT = 14
QUART_SHIFT = 12
TGRID = -(-NROWS // TCOLS)
G_ROWS = TGRID * QUART


def _tc_repack(tt_ref, out_ref):
    x = tt_ref[...]
    u = lax.bitcast_convert_type(x, jnp.int32)
    ur = u + jnp.int32(0x8000)
    lo = (lax.slice(ur, (0, 0), (32, TCOLS)) >> 16) & jnp.int32(0xFFFF)
    hi = lax.slice(ur, (32, 0), (64, TCOLS)) & jnp.int32(-65536)
    w = lo | hi
    w4 = jnp.concatenate(
        [lax.slice(w, (0, q * QUART), (32, (q + 1) * QUART))
         for q in range(4)], axis=0)
    out_ref[...] = lax.transpose(w4, (1, 0))


def _repack_table(tablet):
    return pl.pallas_call(
        _tc_repack,
        grid=(TGRID,),
        in_specs=[pl.BlockSpec((EMBED_DIM, TCOLS), lambda i: (0, i))],
        out_specs=pl.BlockSpec((QUART, PAIR_DIM), lambda i: (i, 0)),
        out_shape=jax.ShapeDtypeStruct((G_ROWS, PAIR_DIM), jnp.int32),
        compiler_params=pltpu.CompilerParams(
            dimension_semantics=("arbitrary",)),
    )(tablet)


def _rsqrt(x):
    i = plsc.bitcast(x, jnp.int32)
    y = plsc.bitcast(jnp.int32(0x5F3759DF) - (i >> 1), jnp.float32)
    for _ in range(3):
        y = y * (jnp.float32(1.5) - jnp.float32(0.5) * x * y * y)
    return y


def _gather_descs(table_hbm, hi_v, rows_v, gsem):
    for k in range(N_ICHUNKS):
        yield pltpu.make_async_copy(
            table_hbm.at[hi_v.at[k]],
            rows_v.at[pl.ds(k * ICHUNK, ICHUNK)], gsem)


def _sc_body(table_hbm, nodes_hbm, out_hbm, idx_v, hi_v, rows_v, t_v, gsem):
    wid = lax.axis_index("s") * NUM_CORES + lax.axis_index("c")
    base = wid * B_PER_W

    pltpu.sync_copy(nodes_hbm.at[wid], idx_v)
    for k in range(N_ICHUNKS):
        for j in range(ICHUNK // LANES):
            sl = pl.ds(j * LANES, LANES)
            iv = idx_v[k, sl]
            hi_v[k, sl] = ((iv >> BLK_SHIFT) << QUART_SHIFT) + \
                (iv & (QUART - 1))

    for c in _gather_descs(table_hbm, hi_v, rows_v, gsem):
        c.start()
    for c in _gather_descs(table_hbm, hi_v, rows_v, gsem):
        c.wait()

    lane = lax.broadcasted_iota(jnp.int32, (LANES,), 0)
    himask = jnp.int32(-65536)

    def chunk_body(c):
        row = c * CCHUNK + lane
        iv = plsc.load_gather(idx_v, [row >> 7, row & (ICHUNK - 1)])
        wb = ((iv >> QUART_SHIFT) & 3) * 32
        acc = jnp.zeros((LANES,), jnp.float32)
        for dp in range(EMBED_DIM // 2):
            w = plsc.load_gather(rows_v, [row, wb + dp])
            ve = plsc.bitcast(w << 16, jnp.float32)
            vo = plsc.bitcast(w & himask, jnp.float32)
            acc = acc + ve * ve + vo * vo
            t_v[dp, pl.ds(c * CCHUNK, CCHUNK)] = ve
            t_v[dp + 32, pl.ds(c * CCHUNK, CCHUNK)] = vo
        r = _rsqrt(acc)
        for d in range(EMBED_DIM):
            sl = pl.ds(c * CCHUNK, CCHUNK)
            t_v[d, sl] = t_v[d, sl] * r

    pl.loop(0, N_CCHUNKS)(chunk_body)

    pltpu.sync_copy(t_v, out_hbm.at[:, pl.ds(base, B_PER_W)])


@jax.jit
def _encode(nodes, table):
    nodes_r = nodes.astype(jnp.int32).reshape(NW, N_ICHUNKS, ICHUNK)
    table_p = _repack_table(table.T)
    mesh = plsc.VectorSubcoreMesh(core_axis_name="c", subcore_axis_name="s")
    return pl.kernel(
        _sc_body,
        out_type=jax.ShapeDtypeStruct((EMBED_DIM, BATCH), jnp.float32),
        mesh=mesh,
        compiler_params=pltpu.CompilerParams(needs_layout_passes=False),
        scratch_types=[
            pltpu.VMEM((N_ICHUNKS, ICHUNK), jnp.int32),
            pltpu.VMEM((N_ICHUNKS, ICHUNK), jnp.int32),
            pltpu.VMEM((B_PER_W, PAIR_DIM), jnp.int32),
            pltpu.VMEM((EMBED_DIM, B_PER_W), jnp.float32),
            pltpu.SemaphoreType.DMA,
        ],
    )(table_p, nodes_r)


def kernel(nodes, table):
    return _encode(nodes, table)
